# Initial kernel scaffold; baseline (speedup 1.0000x reference)
import jax
import jax.numpy as jnp
from jax.experimental import pallas as pl


def kernel(xxinc, xxcord, edgeidx, edgeattr, params):
    raise NotImplementedError("write your pallas kernel here")


# speedup vs baseline: 3.6879x; 3.6879x over previous
import functools

import jax
import jax.numpy as jnp
from jax import lax
from jax.experimental import pallas as pl
from jax.experimental.pallas import tpu as pltpu
from jax.experimental.pallas import tpu_sc as plsc

NC = 2
NS = 16
NW = NC * NS
CHUNK = 128

EDGE_BLOCK = 4096


@functools.partial(jax.jit, static_argnames=("n_chunks",))
def _sc_gather(x, idx3, n_chunks):
  per_tile = n_chunks * CHUNK
  e_pad = NW * per_tile
  mesh = plsc.VectorSubcoreMesh(core_axis_name="c", subcore_axis_name="s")

  @functools.partial(
      pl.kernel,
      out_type=jax.ShapeDtypeStruct((e_pad, 16), jnp.float32),
      mesh=mesh,
      scratch_types=[
          pltpu.VMEM((n_chunks, CHUNK), jnp.int32),
          pltpu.VMEM((per_tile, 16), jnp.float32),
          pltpu.SemaphoreType.DMA,
      ],
      compiler_params=pltpu.CompilerParams(use_tc_tiling_on_sc=False),
  )
  def gather(x_hbm, idx_hbm, out_hbm, idx_v, rows_v, sem):
    wid = lax.axis_index("s") * NC + lax.axis_index("c")
    pltpu.sync_copy(idx_hbm.at[wid], idx_v)

    def fire(j, carry):
      pltpu.make_async_copy(
          x_hbm.at[idx_v.at[j]], rows_v.at[pl.ds(j * CHUNK, CHUNK)], sem
      ).start()
      return carry

    lax.fori_loop(0, n_chunks, fire, 0)

    def drain(j, carry):
      pltpu.make_async_copy(
          x_hbm.at[idx_v.at[j]], rows_v.at[pl.ds(j * CHUNK, CHUNK)], sem
      ).wait()
      return carry

    lax.fori_loop(0, n_chunks, drain, 0)
    pltpu.sync_copy(rows_v, out_hbm.at[pl.ds(wid * per_tile, per_tile)])

  return gather(x, idx3)


@functools.partial(jax.jit, static_argnames=("n_chunks", "n_nodes"))
def _sc_scatter(msg, idx3, n_chunks, n_nodes):
  per_tile = n_chunks * CHUNK
  zrows = -(-(n_nodes + 1) // NS)
  zrows = -(-zrows // 8) * 8
  tbl_rows = zrows * NS
  mesh = plsc.VectorSubcoreMesh(core_axis_name="c", subcore_axis_name="s")

  @functools.partial(
      pl.kernel,
      out_type=jax.ShapeDtypeStruct((NC, tbl_rows, 16), jnp.float32),
      mesh=mesh,
      scratch_types=[
          pltpu.VMEM((n_chunks, CHUNK), jnp.int32),
          pltpu.VMEM((per_tile, 16), jnp.float32),
          pltpu.VMEM((zrows, 16), jnp.float32),
          pltpu.VMEM_SHARED((tbl_rows, 16), jnp.float32),
          pltpu.SemaphoreType.DMA,
      ],
      compiler_params=pltpu.CompilerParams(use_tc_tiling_on_sc=False),
  )
  def scatter(msg_hbm, idx_hbm, out_hbm, idx_v, msg_v, zero_v, tbl, sem):
    cid = lax.axis_index("c")
    sid = lax.axis_index("s")
    wid = sid * NC + cid

    def zbody(i, carry):
      zero_v[i, :] = jnp.zeros((16,), jnp.float32)
      return carry

    lax.fori_loop(0, zrows, zbody, 0)
    pltpu.sync_copy(zero_v, tbl.at[pl.ds(sid * zrows, zrows)])
    pltpu.sync_copy(idx_hbm.at[wid], idx_v)
    pltpu.make_async_copy(
        msg_hbm.at[pl.ds(wid * per_tile, per_tile)], msg_v, sem
    ).start()
    plsc.subcore_barrier()
    pltpu.make_async_copy(
        msg_hbm.at[pl.ds(wid * per_tile, per_tile)], msg_v, sem
    ).wait()

    def sbody(j, carry):
      pltpu.sync_copy(
          msg_v.at[pl.ds(j * CHUNK, CHUNK)], tbl.at[idx_v.at[j]], add=True
      )
      return carry

    lax.fori_loop(0, n_chunks, sbody, 0)
    plsc.subcore_barrier()
    pltpu.sync_copy(
        tbl.at[pl.ds(sid * zrows, zrows)],
        out_hbm.at[cid, pl.ds(sid * zrows, zrows)],
    )

  return scatter(msg, idx3)


def _prelu(x, a):
  return jnp.where(x >= 0, x, a * x)


def _edge_body(ea_t, xs, w1, b1, w2, b2, w3, b3, w4, b4, aa, msg):
  b = xs.shape[0]
  h = jnp.dot(w1[...], ea_t[...], preferred_element_type=jnp.float32) + b1[...]
  h = _prelu(h, aa[0])
  h = jnp.dot(w2[...], h, preferred_element_type=jnp.float32) + b2[...]
  h = _prelu(h, aa[1])
  h = jnp.dot(w3[...], h, preferred_element_type=jnp.float32) + b3[...]
  h = _prelu(h, aa[2])
  z = jnp.dot(w4[...], h, preferred_element_type=jnp.float32) + b4[...]
  xs_t = xs[...].T
  zz = z.reshape(16, 16, b)
  msg_t = jnp.sum(zz * xs_t[:, None, :], axis=0)
  msg[...] = msg_t.T


def _edge_msg(ea_t, xs, w1, b1, w2, b2, w3, b3, w4, b4, aa):
  e_pad = xs.shape[0]
  grid = e_pad // EDGE_BLOCK
  bl = EDGE_BLOCK
  full = lambda j: (0, 0)
  return pl.pallas_call(
      _edge_body,
      grid=(grid,),
      in_specs=[
          pl.BlockSpec((16, bl), lambda j: (0, j)),
          pl.BlockSpec((bl, 16), lambda j: (j, 0)),
          pl.BlockSpec((64, 16), full),
          pl.BlockSpec((64, 1), full),
          pl.BlockSpec((64, 64), full),
          pl.BlockSpec((64, 1), full),
          pl.BlockSpec((64, 64), full),
          pl.BlockSpec((64, 1), full),
          pl.BlockSpec((256, 64), full),
          pl.BlockSpec((256, 1), full),
          pl.BlockSpec(memory_space=pltpu.SMEM),
      ],
      out_specs=pl.BlockSpec((bl, 16), lambda j: (j, 0)),
      out_shape=jax.ShapeDtypeStruct((e_pad, 16), jnp.float32),
      compiler_params=pltpu.CompilerParams(
          dimension_semantics=("parallel",)
      ),
  )(ea_t, xs, w1, b1, w2, b2, w3, b3, w4, b4, aa)


def _update_body(parts, icnt, x, root, bias, aa, out):
  n = x.shape[0]
  mean = (parts[0, :n, :] + parts[1, :n, :]) * icnt[...]
  v = mean + jnp.dot(x[...], root[...], preferred_element_type=jnp.float32)
  v = v + bias[...]
  out[...] = _prelu(v, aa[0])


def _update(parts, icnt, x, root, bias, aa):
  n = x.shape[0]
  return pl.pallas_call(
      _update_body,
      in_specs=[
          pl.BlockSpec(memory_space=pltpu.VMEM),
          pl.BlockSpec(memory_space=pltpu.VMEM),
          pl.BlockSpec(memory_space=pltpu.VMEM),
          pl.BlockSpec(memory_space=pltpu.VMEM),
          pl.BlockSpec(memory_space=pltpu.VMEM),
          pl.BlockSpec(memory_space=pltpu.SMEM),
      ],
      out_specs=pl.BlockSpec(memory_space=pltpu.VMEM),
      out_shape=jax.ShapeDtypeStruct((n, 16), jnp.float32),
  )(parts, icnt, x, root, bias, aa)


def _up_body(xin, w1, b1, w2, b2, w3, b3, aa, cparts, x0, icnt):
  h = jnp.dot(xin[...], w1[...], preferred_element_type=jnp.float32) + b1[...]
  h = _prelu(h, aa[0])
  h = jnp.dot(h, w2[...], preferred_element_type=jnp.float32) + b2[...]
  h = _prelu(h, aa[1])
  x0[...] = jnp.dot(h, w3[...], preferred_element_type=jnp.float32) + b3[...]
  n = xin.shape[0]
  cnt = jnp.maximum(cparts[0, :n, :] + cparts[1, :n, :], 1.0)
  icnt[...] = 1.0 / cnt


def _up(xin, w1, b1, w2, b2, w3, b3, aa, cparts):
  n = xin.shape[0]
  vm = pl.BlockSpec(memory_space=pltpu.VMEM)
  return pl.pallas_call(
      _up_body,
      in_specs=[vm, vm, vm, vm, vm, vm, vm,
                pl.BlockSpec(memory_space=pltpu.SMEM), vm],
      out_specs=(vm, vm),
      out_shape=(
          jax.ShapeDtypeStruct((n, 16), jnp.float32),
          jax.ShapeDtypeStruct((n, 16), jnp.float32),
      ),
  )(xin, w1, b1, w2, b2, w3, b3, aa, cparts)


def _heads_body(x, w1, b1, a1, w2, b2, a2, w3, b3, out):
  h = jnp.dot(x[...], w1[...], preferred_element_type=jnp.float32) + b1[...]
  h = jnp.where(h >= 0, h, a1[...] * h)
  h = jnp.dot(h, w2[...], preferred_element_type=jnp.float32) + b2[...]
  h = jnp.where(h >= 0, h, a2[...] * h)
  out[...] = jnp.dot(h, w3[...], preferred_element_type=jnp.float32) + b3[...]


def _heads(x, w1, b1, a1, w2, b2, a2, w3, b3):
  n = x.shape[0]
  vm = pl.BlockSpec(memory_space=pltpu.VMEM)
  return pl.pallas_call(
      _heads_body,
      in_specs=[vm] * 9,
      out_specs=vm,
      out_shape=jax.ShapeDtypeStruct((n, 6), jnp.float32),
  )(x, w1, b1, a1, w2, b2, a2, w3, b3)


def _block_diag(mats):
  rows = sum(m.shape[0] for m in mats)
  cols = sum(m.shape[1] for m in mats)
  out = jnp.zeros((rows, cols), jnp.float32)
  r = c = 0
  for m in mats:
    out = lax.dynamic_update_slice(out, m, (r, c))
    r += m.shape[0]
    c += m.shape[1]
  return out


def kernel(xxinc, xxcord, edgeidx, edgeattr, params):
  n = xxinc.shape[0]
  e = edgeattr.shape[0]
  per_tile_quantum = NW * CHUNK
  e_pad = -(-e // per_tile_quantum) * per_tile_quantum
  n_chunks = e_pad // (NW * CHUNK)

  src = edgeidx[0].astype(jnp.int32)
  dst = edgeidx[1].astype(jnp.int32)
  pad = e_pad - e
  src3 = jnp.concatenate([src, jnp.zeros((pad,), jnp.int32)]).reshape(
      NW, n_chunks, CHUNK)
  dst3 = jnp.concatenate([dst, jnp.full((pad,), n, jnp.int32)]).reshape(
      NW, n_chunks, CHUNK)

  ea_t = jnp.concatenate(
      [edgeattr, jnp.zeros((pad, edgeattr.shape[1]), jnp.float32)]).T

  ones_msg = jnp.ones((e_pad, 16), jnp.float32)
  cparts = _sc_scatter(ones_msg, dst3, n_chunks, n)

  up = params["up"]
  xin = jnp.concatenate([xxinc, xxcord], axis=1)
  up_aa = jnp.stack([up["a"][0], up["a"][1]])
  x, icnt = _up(
      xin,
      up["lin"][0]["W"], up["lin"][0]["b"][None, :],
      up["lin"][1]["W"], up["lin"][1]["b"][None, :],
      up["lin"][2]["W"], up["lin"][2]["b"][None, :],
      up_aa, cparts)

  for s in params["steps"]:
    aggr = s["aggr"]
    aa = jnp.stack([s["aggr_a"][0], s["aggr_a"][1], s["aggr_a"][2]])
    xs = _sc_gather(x, src3, n_chunks)
    msg = _edge_msg(
        ea_t, xs,
        aggr[0]["W"].T, aggr[0]["b"][:, None],
        aggr[1]["W"].T, aggr[1]["b"][:, None],
        aggr[2]["W"].T, aggr[2]["b"][:, None],
        aggr[3]["W"].T, aggr[3]["b"][:, None],
        aa)
    parts = _sc_scatter(msg, dst3, n_chunks, n)
    x = _update(parts, icnt, x, s["root"], s["bias"][None, :],
                jnp.stack([s["out_a"]]))

  heads = params["heads"]
  w1 = jnp.concatenate([h["lin"][0]["W"] for h in heads], axis=1)
  b1 = jnp.concatenate([h["lin"][0]["b"] for h in heads])[None, :]
  a1 = jnp.concatenate(
      [jnp.full((8,), 1.0) * h["a"][0] for h in heads])[None, :]
  w2 = _block_diag([h["lin"][1]["W"] for h in heads])
  b2 = jnp.concatenate([h["lin"][1]["b"] for h in heads])[None, :]
  a2 = jnp.concatenate(
      [jnp.full((4,), 1.0) * h["a"][1] for h in heads])[None, :]
  w3 = _block_diag([h["lin"][2]["W"] for h in heads])
  b3 = jnp.concatenate([h["lin"][2]["b"] for h in heads])[None, :]
  return _heads(x, w1, b1, a1, w2, b2, a2, w3, b3)


# speedup vs baseline: 5.7986x; 1.5723x over previous
import functools

import jax
import jax.numpy as jnp
from jax import lax
from jax.experimental import pallas as pl
from jax.experimental.pallas import tpu as pltpu
from jax.experimental.pallas import tpu_sc as plsc

NC = 2
NS = 16
NW = NC * NS
CHUNK = 128
RING = 8

EDGE_BLOCK = 4096


@functools.partial(jax.jit, static_argnames=("n_chunks",))
def _sc_gather(x, idx3, n_chunks):
  per_tile = n_chunks * CHUNK
  per_pk = per_tile // 8
  e_pad = NW * per_tile
  cpk = CHUNK // 8
  mesh = plsc.VectorSubcoreMesh(core_axis_name="c", subcore_axis_name="s")

  @functools.partial(
      pl.kernel,
      out_type=jax.ShapeDtypeStruct((e_pad // 8, 128), jnp.float32),
      mesh=mesh,
      scratch_types=[
          pltpu.VMEM((n_chunks, CHUNK), jnp.int32),
          pltpu.VMEM((RING * CHUNK, 16), jnp.float32),
          pltpu.VMEM((per_pk, 128), jnp.float32),
          pltpu.SemaphoreType.DMA,
      ],
      compiler_params=pltpu.CompilerParams(use_tc_tiling_on_sc=False),
  )
  def gather(x_hbm, idx_hbm, out_hbm, idx_v, ring_v, pk_v, sem):
    wid = lax.axis_index("s") * NC + lax.axis_index("c")
    pltpu.sync_copy(idx_hbm.at[wid], idx_v)

    for j in range(RING):
      pltpu.make_async_copy(
          x_hbm.at[idx_v.at[j]], ring_v.at[pl.ds(j * CHUNK, CHUNK)], sem
      ).start()

    def body(j, carry):
      slot = lax.rem(j, RING)
      pltpu.make_async_copy(
          x_hbm.at[idx_v.at[j]], ring_v.at[pl.ds(slot * CHUNK, CHUNK)], sem
      ).wait()

      def rp(rr, c2):
        rbase = slot * CHUNK + 8 * rr
        for q in range(8):
          pk_v[j * cpk + rr, 16 * q:16 * (q + 1)] = ring_v[rbase + q, :]
        return c2

      lax.fori_loop(0, cpk, rp, 0)

      @pl.when(j + RING < n_chunks)
      def _():
        pltpu.make_async_copy(
            x_hbm.at[idx_v.at[j + RING]],
            ring_v.at[pl.ds(slot * CHUNK, CHUNK)], sem).start()

      return carry

    lax.fori_loop(0, n_chunks, body, 0)
    pltpu.sync_copy(pk_v, out_hbm.at[pl.ds(wid * per_pk, per_pk)])

  return gather(x, idx3)


@functools.partial(jax.jit, static_argnames=("n_chunks", "n_nodes"))
def _sc_scatter(msg_pk, idx3, n_chunks, n_nodes):
  per_tile = n_chunks * CHUNK
  per_pk = per_tile // 8
  cpk = CHUNK // 8
  zrows = -(-(n_nodes + 1) // NS)
  zrows = -(-zrows // 8) * 8
  tbl_rows = zrows * NS
  mesh = plsc.VectorSubcoreMesh(core_axis_name="c", subcore_axis_name="s")

  @functools.partial(
      pl.kernel,
      out_type=jax.ShapeDtypeStruct((NC, tbl_rows, 16), jnp.float32),
      mesh=mesh,
      scratch_types=[
          pltpu.VMEM((n_chunks, CHUNK), jnp.int32),
          pltpu.VMEM((per_pk, 128), jnp.float32),
          pltpu.VMEM((CHUNK, 16), jnp.float32),
          pltpu.VMEM((zrows, 16), jnp.float32),
          pltpu.VMEM_SHARED((tbl_rows, 16), jnp.float32),
          pltpu.SemaphoreType.DMA,
      ],
      compiler_params=pltpu.CompilerParams(use_tc_tiling_on_sc=False),
  )
  def scatter(msg_hbm, idx_hbm, out_hbm, idx_v, pk_v, grp_v, row_v, tbl,
              sem):
    cid = lax.axis_index("c")
    sid = lax.axis_index("s")
    wid = sid * NC + cid

    pltpu.make_async_copy(
        msg_hbm.at[pl.ds(wid * per_pk, per_pk)], pk_v, sem).start()

    def zbody(i, carry):
      row_v[i, :] = jnp.zeros((16,), jnp.float32)
      return carry

    lax.fori_loop(0, zrows, zbody, 0)
    pltpu.sync_copy(row_v, tbl.at[pl.ds(sid * zrows, zrows)])
    pltpu.sync_copy(idx_hbm.at[wid], idx_v)
    plsc.subcore_barrier()
    pltpu.make_async_copy(
        msg_hbm.at[pl.ds(wid * per_pk, per_pk)], pk_v, sem).wait()

    def sbody(g, carry):
      def unpack(rr, c2):
        for q in range(8):
          grp_v[8 * rr + q, :] = pk_v[g * cpk + rr, 16 * q:16 * (q + 1)]
        return c2

      lax.fori_loop(0, cpk, unpack, 0)
      pltpu.sync_copy(grp_v, tbl.at[idx_v.at[g]], add=True)
      return carry

    lax.fori_loop(0, n_chunks, sbody, 0)
    plsc.subcore_barrier()
    pltpu.sync_copy(
        tbl.at[pl.ds(sid * zrows, zrows)],
        out_hbm.at[cid, pl.ds(sid * zrows, zrows)],
    )

  return scatter(msg_pk, idx3)


def _prelu(x, a):
  return jnp.where(x >= 0, x, a * x)


def _edge_body(ea_t, xs_pk, w1, b1, w2, b2, w3, b3, w4, b4, aa, msg_pk):
  b = xs_pk.shape[0] * 8
  r = b // 8
  h = jnp.dot(w1[...], ea_t[...], preferred_element_type=jnp.float32) + b1[...]
  h = _prelu(h, aa[0])
  h = jnp.dot(w2[...], h, preferred_element_type=jnp.float32) + b2[...]
  h = _prelu(h, aa[1])
  h = jnp.dot(w3[...], h, preferred_element_type=jnp.float32) + b3[...]
  h = _prelu(h, aa[2])
  z = jnp.dot(w4[...], h, preferred_element_type=jnp.float32) + b4[...]
  xt = xs_pk[...].T
  xs_t = jnp.concatenate(
      [xt[16 * q:16 * (q + 1), :] for q in range(8)], axis=1)
  zz = z.reshape(16, 16, b)
  msg_t = jnp.sum(zz * xs_t[:, None, :], axis=0)
  m128 = jnp.concatenate(
      [msg_t[:, r * q:r * (q + 1)] for q in range(8)], axis=0)
  msg_pk[...] = m128.T


def _edge_msg(ea_t, xs_pk, w1, b1, w2, b2, w3, b3, w4, b4, aa):
  e_pad = xs_pk.shape[0] * 8
  grid = e_pad // EDGE_BLOCK
  bl = EDGE_BLOCK
  full = lambda j: (0, 0)
  return pl.pallas_call(
      _edge_body,
      grid=(grid,),
      in_specs=[
          pl.BlockSpec((16, bl), lambda j: (0, j)),
          pl.BlockSpec((bl // 8, 128), lambda j: (j, 0)),
          pl.BlockSpec((64, 16), full),
          pl.BlockSpec((64, 1), full),
          pl.BlockSpec((64, 64), full),
          pl.BlockSpec((64, 1), full),
          pl.BlockSpec((64, 64), full),
          pl.BlockSpec((64, 1), full),
          pl.BlockSpec((256, 64), full),
          pl.BlockSpec((256, 1), full),
          pl.BlockSpec(memory_space=pltpu.SMEM),
      ],
      out_specs=pl.BlockSpec((bl // 8, 128), lambda j: (j, 0)),
      out_shape=jax.ShapeDtypeStruct((e_pad // 8, 128), jnp.float32),
      compiler_params=pltpu.CompilerParams(
          dimension_semantics=("parallel",)
      ),
  )(ea_t, xs_pk, w1, b1, w2, b2, w3, b3, w4, b4, aa)


def _update_body(parts, icnt, x, root, bias, aa, out):
  n = x.shape[0]
  mean = (parts[0, :n, :] + parts[1, :n, :]) * icnt[...]
  v = mean + jnp.dot(x[...], root[...], preferred_element_type=jnp.float32)
  v = v + bias[...]
  out[...] = _prelu(v, aa[0])


def _update(parts, icnt, x, root, bias, aa):
  n = x.shape[0]
  vm = pl.BlockSpec(memory_space=pltpu.VMEM)
  return pl.pallas_call(
      _update_body,
      in_specs=[vm, vm, vm, vm, vm,
                pl.BlockSpec(memory_space=pltpu.SMEM)],
      out_specs=vm,
      out_shape=jax.ShapeDtypeStruct((n, 16), jnp.float32),
  )(parts, icnt, x, root, bias, aa)


def _up_body(xin, w1, b1, w2, b2, w3, b3, aa, cparts, x0, icnt):
  h = jnp.dot(xin[...], w1[...], preferred_element_type=jnp.float32) + b1[...]
  h = _prelu(h, aa[0])
  h = jnp.dot(h, w2[...], preferred_element_type=jnp.float32) + b2[...]
  h = _prelu(h, aa[1])
  x0[...] = jnp.dot(h, w3[...], preferred_element_type=jnp.float32) + b3[...]
  n = xin.shape[0]
  cnt = jnp.maximum(cparts[0, :n, :] + cparts[1, :n, :], 1.0)
  icnt[...] = 1.0 / cnt


def _up(xin, w1, b1, w2, b2, w3, b3, aa, cparts):
  n = xin.shape[0]
  vm = pl.BlockSpec(memory_space=pltpu.VMEM)
  return pl.pallas_call(
      _up_body,
      in_specs=[vm, vm, vm, vm, vm, vm, vm,
                pl.BlockSpec(memory_space=pltpu.SMEM), vm],
      out_specs=(vm, vm),
      out_shape=(
          jax.ShapeDtypeStruct((n, 16), jnp.float32),
          jax.ShapeDtypeStruct((n, 16), jnp.float32),
      ),
  )(xin, w1, b1, w2, b2, w3, b3, aa, cparts)


def _heads_body(x, w1, b1, a1, w2, b2, a2, w3, b3, out):
  h = jnp.dot(x[...], w1[...], preferred_element_type=jnp.float32) + b1[...]
  h = jnp.where(h >= 0, h, a1[...] * h)
  h = jnp.dot(h, w2[...], preferred_element_type=jnp.float32) + b2[...]
  h = jnp.where(h >= 0, h, a2[...] * h)
  out[...] = jnp.dot(h, w3[...], preferred_element_type=jnp.float32) + b3[...]


def _heads(x, w1, b1, a1, w2, b2, a2, w3, b3):
  n = x.shape[0]
  vm = pl.BlockSpec(memory_space=pltpu.VMEM)
  return pl.pallas_call(
      _heads_body,
      in_specs=[vm] * 9,
      out_specs=vm,
      out_shape=jax.ShapeDtypeStruct((n, 6), jnp.float32),
  )(x, w1, b1, a1, w2, b2, a2, w3, b3)


def _block_diag(mats):
  rows = sum(m.shape[0] for m in mats)
  cols = sum(m.shape[1] for m in mats)
  out = jnp.zeros((rows, cols), jnp.float32)
  r = c = 0
  for m in mats:
    out = lax.dynamic_update_slice(out, m, (r, c))
    r += m.shape[0]
    c += m.shape[1]
  return out


def kernel(xxinc, xxcord, edgeidx, edgeattr, params):
  n = xxinc.shape[0]
  e = edgeattr.shape[0]
  per_tile_quantum = NW * CHUNK
  e_pad = -(-e // per_tile_quantum) * per_tile_quantum
  n_chunks = e_pad // (NW * CHUNK)

  src = edgeidx[0].astype(jnp.int32)
  dst = edgeidx[1].astype(jnp.int32)
  pad = e_pad - e
  src3 = jnp.concatenate([src, jnp.zeros((pad,), jnp.int32)]).reshape(
      NW, n_chunks, CHUNK)
  dst3 = jnp.concatenate([dst, jnp.full((pad,), n, jnp.int32)]).reshape(
      NW, n_chunks, CHUNK)

  bl = EDGE_BLOCK
  rpb = bl // 8
  p_arr = jnp.arange(e_pad, dtype=jnp.int32)
  bblk = p_arr // bl
  lam = p_arr % bl
  iperm = bblk * bl + 8 * (lam % rpb) + lam // rpb
  ea_pad = jnp.concatenate(
      [edgeattr, jnp.zeros((pad, edgeattr.shape[1]), jnp.float32)])
  ea_t = ea_pad[iperm, :].T

  ones_msg = jnp.ones((e_pad // 8, 128), jnp.float32)
  cparts = _sc_scatter(ones_msg, dst3, n_chunks, n)

  up = params["up"]
  xin = jnp.concatenate([xxinc, xxcord], axis=1)
  up_aa = jnp.stack([up["a"][0], up["a"][1]])
  x, icnt = _up(
      xin,
      up["lin"][0]["W"], up["lin"][0]["b"][None, :],
      up["lin"][1]["W"], up["lin"][1]["b"][None, :],
      up["lin"][2]["W"], up["lin"][2]["b"][None, :],
      up_aa, cparts)

  for s in params["steps"]:
    aggr = s["aggr"]
    aa = jnp.stack([s["aggr_a"][0], s["aggr_a"][1], s["aggr_a"][2]])
    xs_tb = _sc_gather(x, src3, n_chunks)
    msg_tb = _edge_msg(
        ea_t, xs_tb,
        aggr[0]["W"].T, aggr[0]["b"][:, None],
        aggr[1]["W"].T, aggr[1]["b"][:, None],
        aggr[2]["W"].T, aggr[2]["b"][:, None],
        aggr[3]["W"].T, aggr[3]["b"][:, None],
        aa)
    parts = _sc_scatter(msg_tb, dst3, n_chunks, n)
    x = _update(parts, icnt, x, s["root"], s["bias"][None, :],
                jnp.stack([s["out_a"]]))

  heads = params["heads"]
  w1 = jnp.concatenate([h["lin"][0]["W"] for h in heads], axis=1)
  b1 = jnp.concatenate([h["lin"][0]["b"] for h in heads])[None, :]
  a1 = jnp.concatenate(
      [jnp.full((8,), 1.0) * h["a"][0] for h in heads])[None, :]
  w2 = _block_diag([h["lin"][1]["W"] for h in heads])
  b2 = jnp.concatenate([h["lin"][1]["b"] for h in heads])[None, :]
  a2 = jnp.concatenate(
      [jnp.full((4,), 1.0) * h["a"][1] for h in heads])[None, :]
  w3 = _block_diag([h["lin"][2]["W"] for h in heads])
  b3 = jnp.concatenate([h["lin"][2]["b"] for h in heads])[None, :]
  return _heads(x, w1, b1, a1, w2, b2, a2, w3, b3)


# speedup vs baseline: 6.8587x; 1.1828x over previous
import functools

import jax
import jax.numpy as jnp
from jax import lax
from jax.experimental import pallas as pl
from jax.experimental.pallas import tpu as pltpu
from jax.experimental.pallas import tpu_sc as plsc

NC = 2
NS = 16
NW = NC * NS
CHUNK = 128
RING = 8

EDGE_BLOCK = 4096


@functools.partial(jax.jit, static_argnames=("n_chunks",))
def _sc_gather(x, idx3, n_chunks):
  per_tile = n_chunks * CHUNK
  per_pk = per_tile // 8
  e_pad = NW * per_tile
  cpk = CHUNK // 8
  mesh = plsc.VectorSubcoreMesh(core_axis_name="c", subcore_axis_name="s")

  @functools.partial(
      pl.kernel,
      out_type=jax.ShapeDtypeStruct((e_pad // 8, 128), jnp.float32),
      mesh=mesh,
      scratch_types=[
          pltpu.VMEM((n_chunks, CHUNK), jnp.int32),
          pltpu.VMEM((RING * CHUNK, 16), jnp.float32),
          pltpu.VMEM((per_pk, 128), jnp.float32),
          pltpu.SemaphoreType.DMA,
      ],
      compiler_params=pltpu.CompilerParams(use_tc_tiling_on_sc=False),
  )
  def gather(x_hbm, idx_hbm, out_hbm, idx_v, ring_v, pk_v, sem):
    wid = lax.axis_index("s") * NC + lax.axis_index("c")
    pltpu.sync_copy(idx_hbm.at[wid], idx_v)

    for j in range(RING):
      pltpu.make_async_copy(
          x_hbm.at[idx_v.at[j]], ring_v.at[pl.ds(j * CHUNK, CHUNK)], sem
      ).start()

    def body(j, carry):
      slot = lax.rem(j, RING)
      pltpu.make_async_copy(
          x_hbm.at[idx_v.at[j]], ring_v.at[pl.ds(slot * CHUNK, CHUNK)], sem
      ).wait()

      def rp(rr, c2):
        rbase = slot * CHUNK + 8 * rr
        for q in range(8):
          pk_v[j * cpk + rr, 16 * q:16 * (q + 1)] = ring_v[rbase + q, :]
        return c2

      lax.fori_loop(0, cpk, rp, 0)

      @pl.when(j + RING < n_chunks)
      def _():
        pltpu.make_async_copy(
            x_hbm.at[idx_v.at[j + RING]],
            ring_v.at[pl.ds(slot * CHUNK, CHUNK)], sem).start()

      return carry

    lax.fori_loop(0, n_chunks, body, 0)
    pltpu.sync_copy(pk_v, out_hbm.at[pl.ds(wid * per_pk, per_pk)])

  return gather(x, idx3)


@functools.partial(jax.jit, static_argnames=("n_chunks", "n_nodes"))
def _sc_scatter(msg_pk, idx3, n_chunks, n_nodes):
  per_tile = n_chunks * CHUNK
  per_pk = per_tile // 8
  cpk = CHUNK // 8
  zrows = -(-(n_nodes + 1) // NS)
  zrows = -(-zrows // 8) * 8
  tbl_rows = zrows * NS
  mesh = plsc.VectorSubcoreMesh(core_axis_name="c", subcore_axis_name="s")

  @functools.partial(
      pl.kernel,
      out_type=jax.ShapeDtypeStruct((NC, tbl_rows, 16), jnp.float32),
      mesh=mesh,
      scratch_types=[
          pltpu.VMEM((n_chunks, CHUNK), jnp.int32),
          pltpu.VMEM((per_pk, 128), jnp.float32),
          pltpu.VMEM((CHUNK, 16), jnp.float32),
          pltpu.VMEM((zrows, 16), jnp.float32),
          pltpu.VMEM_SHARED((tbl_rows, 16), jnp.float32),
          pltpu.SemaphoreType.DMA,
      ],
      compiler_params=pltpu.CompilerParams(use_tc_tiling_on_sc=False),
  )
  def scatter(msg_hbm, idx_hbm, out_hbm, idx_v, pk_v, grp_v, row_v, tbl,
              sem):
    cid = lax.axis_index("c")
    sid = lax.axis_index("s")
    wid = sid * NC + cid

    pltpu.make_async_copy(
        msg_hbm.at[pl.ds(wid * per_pk, per_pk)], pk_v, sem).start()

    def zbody(i, carry):
      row_v[i, :] = jnp.zeros((16,), jnp.float32)
      return carry

    lax.fori_loop(0, zrows, zbody, 0)
    pltpu.sync_copy(row_v, tbl.at[pl.ds(sid * zrows, zrows)])
    pltpu.sync_copy(idx_hbm.at[wid], idx_v)
    plsc.subcore_barrier()
    pltpu.make_async_copy(
        msg_hbm.at[pl.ds(wid * per_pk, per_pk)], pk_v, sem).wait()

    def sbody(g, carry):
      def unpack(rr, c2):
        for q in range(8):
          grp_v[8 * rr + q, :] = pk_v[g * cpk + rr, 16 * q:16 * (q + 1)]
        return c2

      lax.fori_loop(0, cpk, unpack, 0)
      pltpu.sync_copy(grp_v, tbl.at[idx_v.at[g]], add=True)
      return carry

    lax.fori_loop(0, n_chunks, sbody, 0)
    plsc.subcore_barrier()
    pltpu.sync_copy(
        tbl.at[pl.ds(sid * zrows, zrows)],
        out_hbm.at[cid, pl.ds(sid * zrows, zrows)],
    )

  return scatter(msg_pk, idx3)


def _prelu(x, a):
  return jnp.where(x >= 0, x, a * x)


def _edge_body(ea_t, xs_pk, w1, b1, w2, b2, w3, b3, w4, b4, aa, msg_pk):
  b = xs_pk.shape[0] * 8
  r = b // 8
  h = jnp.dot(w1[...], ea_t[...], preferred_element_type=jnp.float32) + b1[...]
  h = _prelu(h, aa[0])
  h = jnp.dot(w2[...], h, preferred_element_type=jnp.float32) + b2[...]
  h = _prelu(h, aa[1])
  h = jnp.dot(w3[...], h, preferred_element_type=jnp.float32) + b3[...]
  h = _prelu(h, aa[2])
  z = jnp.dot(w4[...], h, preferred_element_type=jnp.float32) + b4[...]
  xt = xs_pk[...].T
  xs_t = jnp.concatenate(
      [xt[16 * q:16 * (q + 1), :] for q in range(8)], axis=1)
  zz = z.reshape(16, 16, b)
  msg_t = jnp.sum(zz * xs_t[:, None, :], axis=0)
  m128 = jnp.concatenate(
      [msg_t[:, r * q:r * (q + 1)] for q in range(8)], axis=0)
  msg_pk[...] = m128.T


def _edge_msg(ea_t, xs_pk, w1, b1, w2, b2, w3, b3, w4, b4, aa):
  e_pad = xs_pk.shape[0] * 8
  grid = e_pad // EDGE_BLOCK
  bl = EDGE_BLOCK
  full = lambda j: (0, 0)
  return pl.pallas_call(
      _edge_body,
      grid=(grid,),
      in_specs=[
          pl.BlockSpec((16, bl), lambda j: (0, j)),
          pl.BlockSpec((bl // 8, 128), lambda j: (j, 0)),
          pl.BlockSpec((64, 16), full),
          pl.BlockSpec((64, 1), full),
          pl.BlockSpec((64, 64), full),
          pl.BlockSpec((64, 1), full),
          pl.BlockSpec((64, 64), full),
          pl.BlockSpec((64, 1), full),
          pl.BlockSpec((256, 64), full),
          pl.BlockSpec((256, 1), full),
          pl.BlockSpec(memory_space=pltpu.SMEM),
      ],
      out_specs=pl.BlockSpec((bl // 8, 128), lambda j: (j, 0)),
      out_shape=jax.ShapeDtypeStruct((e_pad // 8, 128), jnp.float32),
      compiler_params=pltpu.CompilerParams(
          dimension_semantics=("parallel",)
      ),
  )(ea_t, xs_pk, w1, b1, w2, b2, w3, b3, w4, b4, aa)


def _update_body(pa, pb, icnt, x, root, bias, aa, out):
  n = x.shape[0]
  s = pa[0, :n, :] + pa[1, :n, :] + pb[0, :n, :] + pb[1, :n, :]
  mean = s * icnt[...]
  v = mean + jnp.dot(x[...], root[...], preferred_element_type=jnp.float32)
  v = v + bias[...]
  out[...] = _prelu(v, aa[0])


def _update(pa, pb, icnt, x, root, bias, aa):
  n = x.shape[0]
  vm = pl.BlockSpec(memory_space=pltpu.VMEM)
  return pl.pallas_call(
      _update_body,
      in_specs=[vm, vm, vm, vm, vm, vm,
                pl.BlockSpec(memory_space=pltpu.SMEM)],
      out_specs=vm,
      out_shape=jax.ShapeDtypeStruct((n, 16), jnp.float32),
  )(pa, pb, icnt, x, root, bias, aa)


def _up_body(xin, w1, b1, w2, b2, w3, b3, aa, cparts, x0, icnt):
  h = jnp.dot(xin[...], w1[...], preferred_element_type=jnp.float32) + b1[...]
  h = _prelu(h, aa[0])
  h = jnp.dot(h, w2[...], preferred_element_type=jnp.float32) + b2[...]
  h = _prelu(h, aa[1])
  x0[...] = jnp.dot(h, w3[...], preferred_element_type=jnp.float32) + b3[...]
  n = xin.shape[0]
  cnt = jnp.maximum(
      cparts[0, 0, :n, :] + cparts[0, 1, :n, :]
      + cparts[1, 0, :n, :] + cparts[1, 1, :n, :], 1.0)
  icnt[...] = 1.0 / cnt


def _up(xin, w1, b1, w2, b2, w3, b3, aa, cparts):
  n = xin.shape[0]
  vm = pl.BlockSpec(memory_space=pltpu.VMEM)
  return pl.pallas_call(
      _up_body,
      in_specs=[vm, vm, vm, vm, vm, vm, vm,
                pl.BlockSpec(memory_space=pltpu.SMEM), vm],
      out_specs=(vm, vm),
      out_shape=(
          jax.ShapeDtypeStruct((n, 16), jnp.float32),
          jax.ShapeDtypeStruct((n, 16), jnp.float32),
      ),
  )(xin, w1, b1, w2, b2, w3, b3, aa, cparts)


def _heads_body(x, w1, b1, a1, w2, b2, a2, w3, b3, out):
  h = jnp.dot(x[...], w1[...], preferred_element_type=jnp.float32) + b1[...]
  h = jnp.where(h >= 0, h, a1[...] * h)
  h = jnp.dot(h, w2[...], preferred_element_type=jnp.float32) + b2[...]
  h = jnp.where(h >= 0, h, a2[...] * h)
  out[...] = jnp.dot(h, w3[...], preferred_element_type=jnp.float32) + b3[...]


def _heads(x, w1, b1, a1, w2, b2, a2, w3, b3):
  n = x.shape[0]
  vm = pl.BlockSpec(memory_space=pltpu.VMEM)
  return pl.pallas_call(
      _heads_body,
      in_specs=[vm] * 9,
      out_specs=vm,
      out_shape=jax.ShapeDtypeStruct((n, 6), jnp.float32),
  )(x, w1, b1, a1, w2, b2, a2, w3, b3)


def _block_diag(mats):
  rows = sum(m.shape[0] for m in mats)
  cols = sum(m.shape[1] for m in mats)
  out = jnp.zeros((rows, cols), jnp.float32)
  r = c = 0
  for m in mats:
    out = lax.dynamic_update_slice(out, m, (r, c))
    r += m.shape[0]
    c += m.shape[1]
  return out


def kernel(xxinc, xxcord, edgeidx, edgeattr, params):
  n = xxinc.shape[0]
  e = edgeattr.shape[0]
  per_tile_quantum = 2 * NW * CHUNK
  e_pad = -(-e // per_tile_quantum) * per_tile_quantum
  half = e_pad // 2
  n_chunks = half // (NW * CHUNK)

  src = edgeidx[0].astype(jnp.int32)
  dst = edgeidx[1].astype(jnp.int32)
  pad = e_pad - e
  src_pad = jnp.concatenate([src, jnp.zeros((pad,), jnp.int32)])
  dst_pad = jnp.concatenate([dst, jnp.full((pad,), n, jnp.int32)])
  src3 = [src_pad[i * half:(i + 1) * half].reshape(NW, n_chunks, CHUNK)
          for i in range(2)]
  dst3 = [dst_pad[i * half:(i + 1) * half].reshape(NW, n_chunks, CHUNK)
          for i in range(2)]

  bl = EDGE_BLOCK
  rpb = bl // 8
  ea_pad = jnp.concatenate(
      [edgeattr, jnp.zeros((pad, edgeattr.shape[1]), jnp.float32)])
  ea_perm = ea_pad.reshape(e_pad // bl, rpb, 8, 16).transpose(
      0, 2, 1, 3).reshape(e_pad, 16)
  ea_t = [ea_perm[i * half:(i + 1) * half].T for i in range(2)]

  ones_msg = jnp.ones((half // 8, 128), jnp.float32)
  cparts = jnp.stack([_sc_scatter(ones_msg, dst3[0], n_chunks, n),
                      _sc_scatter(ones_msg, dst3[1], n_chunks, n)])

  up = params["up"]
  xin = jnp.concatenate([xxinc, xxcord], axis=1)
  up_aa = jnp.stack([up["a"][0], up["a"][1]])
  x, icnt = _up(
      xin,
      up["lin"][0]["W"], up["lin"][0]["b"][None, :],
      up["lin"][1]["W"], up["lin"][1]["b"][None, :],
      up["lin"][2]["W"], up["lin"][2]["b"][None, :],
      up_aa, cparts)

  for s in params["steps"]:
    aggr = s["aggr"]
    aa = jnp.stack([s["aggr_a"][0], s["aggr_a"][1], s["aggr_a"][2]])
    ws = (aggr[0]["W"].T, aggr[0]["b"][:, None],
          aggr[1]["W"].T, aggr[1]["b"][:, None],
          aggr[2]["W"].T, aggr[2]["b"][:, None],
          aggr[3]["W"].T, aggr[3]["b"][:, None], aa)
    xs_a = _sc_gather(x, src3[0], n_chunks)
    xs_b = _sc_gather(x, src3[1], n_chunks)
    msg_a = _edge_msg(ea_t[0], xs_a, *ws)
    parts_a = _sc_scatter(msg_a, dst3[0], n_chunks, n)
    msg_b = _edge_msg(ea_t[1], xs_b, *ws)
    parts_b = _sc_scatter(msg_b, dst3[1], n_chunks, n)
    x = _update(parts_a, parts_b, icnt, x, s["root"], s["bias"][None, :],
                jnp.stack([s["out_a"]]))

  heads = params["heads"]
  w1 = jnp.concatenate([h["lin"][0]["W"] for h in heads], axis=1)
  b1 = jnp.concatenate([h["lin"][0]["b"] for h in heads])[None, :]
  a1 = jnp.concatenate(
      [jnp.full((8,), 1.0) * h["a"][0] for h in heads])[None, :]
  w2 = _block_diag([h["lin"][1]["W"] for h in heads])
  b2 = jnp.concatenate([h["lin"][1]["b"] for h in heads])[None, :]
  a2 = jnp.concatenate(
      [jnp.full((4,), 1.0) * h["a"][1] for h in heads])[None, :]
  w3 = _block_diag([h["lin"][2]["W"] for h in heads])
  b3 = jnp.concatenate([h["lin"][2]["b"] for h in heads])[None, :]
  return _heads(x, w1, b1, a1, w2, b2, a2, w3, b3)


# speedup vs baseline: 6.8598x; 1.0002x over previous
import functools

import jax
import jax.numpy as jnp
from jax import lax
from jax.experimental import pallas as pl
from jax.experimental.pallas import tpu as pltpu
from jax.experimental.pallas import tpu_sc as plsc

NC = 2
NS = 16
NW = NC * NS
CHUNK = 128
RING = 8

EDGE_BLOCK = 4096


@functools.partial(jax.jit, static_argnames=("n_chunks",))
def _sc_gather(x, idx3, n_chunks):
  per_tile = n_chunks * CHUNK
  per_pk = per_tile // 8
  e_pad = NW * per_tile
  cpk = CHUNK // 8
  mesh = plsc.VectorSubcoreMesh(core_axis_name="c", subcore_axis_name="s")

  @functools.partial(
      pl.kernel,
      out_type=jax.ShapeDtypeStruct((e_pad // 8, 128), jnp.float32),
      mesh=mesh,
      scratch_types=[
          pltpu.VMEM((n_chunks, CHUNK), jnp.int32),
          pltpu.VMEM((RING * CHUNK, 16), jnp.float32),
          pltpu.VMEM((per_pk, 128), jnp.float32),
          pltpu.SemaphoreType.DMA,
      ],
      compiler_params=pltpu.CompilerParams(use_tc_tiling_on_sc=False),
  )
  def gather(x_hbm, idx_hbm, out_hbm, idx_v, ring_v, pk_v, sem):
    wid = lax.axis_index("s") * NC + lax.axis_index("c")
    pltpu.sync_copy(idx_hbm.at[wid], idx_v)

    for j in range(RING):
      pltpu.make_async_copy(
          x_hbm.at[idx_v.at[j]], ring_v.at[pl.ds(j * CHUNK, CHUNK)], sem
      ).start()

    def body(j, carry):
      slot = lax.rem(j, RING)
      pltpu.make_async_copy(
          x_hbm.at[idx_v.at[j]], ring_v.at[pl.ds(slot * CHUNK, CHUNK)], sem
      ).wait()

      def rp(rr, c2):
        rbase = slot * CHUNK + 8 * rr
        for q in range(8):
          pk_v[j * cpk + rr, 16 * q:16 * (q + 1)] = ring_v[rbase + q, :]
        return c2

      lax.fori_loop(0, cpk, rp, 0)

      @pl.when(j + RING < n_chunks)
      def _():
        pltpu.make_async_copy(
            x_hbm.at[idx_v.at[j + RING]],
            ring_v.at[pl.ds(slot * CHUNK, CHUNK)], sem).start()

      return carry

    lax.fori_loop(0, n_chunks, body, 0)
    pltpu.sync_copy(pk_v, out_hbm.at[pl.ds(wid * per_pk, per_pk)])

  return gather(x, idx3)


@functools.partial(jax.jit, static_argnames=("n_chunks", "n_nodes"))
def _sc_scatter(msg_pk, idx3, n_chunks, n_nodes):
  per_tile = n_chunks * CHUNK
  per_pk = per_tile // 8
  cpk = CHUNK // 8
  zrows = -(-(n_nodes + 1) // NS)
  zrows = -(-zrows // 8) * 8
  tbl_rows = zrows * NS
  mesh = plsc.VectorSubcoreMesh(core_axis_name="c", subcore_axis_name="s")

  @functools.partial(
      pl.kernel,
      out_type=jax.ShapeDtypeStruct((NC, tbl_rows, 16), jnp.float32),
      mesh=mesh,
      scratch_types=[
          pltpu.VMEM((n_chunks, CHUNK), jnp.int32),
          pltpu.VMEM((per_pk, 128), jnp.float32),
          pltpu.VMEM((CHUNK, 16), jnp.float32),
          pltpu.VMEM((zrows, 16), jnp.float32),
          pltpu.VMEM_SHARED((tbl_rows, 16), jnp.float32),
          pltpu.SemaphoreType.DMA,
      ],
      compiler_params=pltpu.CompilerParams(use_tc_tiling_on_sc=False),
  )
  def scatter(msg_hbm, idx_hbm, out_hbm, idx_v, pk_v, grp_v, row_v, tbl,
              sem):
    cid = lax.axis_index("c")
    sid = lax.axis_index("s")
    wid = sid * NC + cid

    pltpu.make_async_copy(
        msg_hbm.at[pl.ds(wid * per_pk, per_pk)], pk_v, sem).start()

    def zbody(i, carry):
      row_v[i, :] = jnp.zeros((16,), jnp.float32)
      return carry

    lax.fori_loop(0, zrows, zbody, 0)
    pltpu.sync_copy(row_v, tbl.at[pl.ds(sid * zrows, zrows)])
    pltpu.sync_copy(idx_hbm.at[wid], idx_v)
    plsc.subcore_barrier()
    pltpu.make_async_copy(
        msg_hbm.at[pl.ds(wid * per_pk, per_pk)], pk_v, sem).wait()

    def sbody(g, carry):
      def unpack(rr, c2):
        for q in range(8):
          grp_v[8 * rr + q, :] = pk_v[g * cpk + rr, 16 * q:16 * (q + 1)]
        return c2

      lax.fori_loop(0, cpk, unpack, 0)
      pltpu.sync_copy(grp_v, tbl.at[idx_v.at[g]], add=True)
      return carry

    lax.fori_loop(0, n_chunks, sbody, 0)
    plsc.subcore_barrier()
    pltpu.sync_copy(
        tbl.at[pl.ds(sid * zrows, zrows)],
        out_hbm.at[cid, pl.ds(sid * zrows, zrows)],
    )

  return scatter(msg_pk, idx3)


def _prelu(x, a):
  return jnp.where(x >= 0, x, a * x)


def _edge_body(ea_t, xs_pk, w1, b1, w2, b2, w3, b3, w4, b4, aa, msg_pk):
  b = xs_pk.shape[0] * 8
  r = b // 8
  bf = jnp.bfloat16
  h = jnp.dot(w1[...].astype(bf), ea_t[...].astype(bf),
              preferred_element_type=jnp.float32) + b1[...]
  h = _prelu(h, aa[0])
  h = jnp.dot(w2[...].astype(bf), h.astype(bf),
              preferred_element_type=jnp.float32) + b2[...]
  h = _prelu(h, aa[1])
  h = jnp.dot(w3[...].astype(bf), h.astype(bf),
              preferred_element_type=jnp.float32) + b3[...]
  h = _prelu(h, aa[2])
  z = jnp.dot(w4[...].astype(bf), h.astype(bf),
              preferred_element_type=jnp.float32) + b4[...]
  xt = xs_pk[...].T
  xs_t = jnp.concatenate(
      [xt[16 * q:16 * (q + 1), :] for q in range(8)], axis=1)
  zz = z.reshape(16, 16, b)
  msg_t = jnp.sum(zz * xs_t[:, None, :], axis=0)
  m128 = jnp.concatenate(
      [msg_t[:, r * q:r * (q + 1)] for q in range(8)], axis=0)
  msg_pk[...] = m128.T


def _edge_msg(ea_t, xs_pk, w1, b1, w2, b2, w3, b3, w4, b4, aa):
  e_pad = xs_pk.shape[0] * 8
  grid = e_pad // EDGE_BLOCK
  bl = EDGE_BLOCK
  full = lambda j: (0, 0)
  return pl.pallas_call(
      _edge_body,
      grid=(grid,),
      in_specs=[
          pl.BlockSpec((16, bl), lambda j: (0, j)),
          pl.BlockSpec((bl // 8, 128), lambda j: (j, 0)),
          pl.BlockSpec((64, 16), full),
          pl.BlockSpec((64, 1), full),
          pl.BlockSpec((64, 64), full),
          pl.BlockSpec((64, 1), full),
          pl.BlockSpec((64, 64), full),
          pl.BlockSpec((64, 1), full),
          pl.BlockSpec((256, 64), full),
          pl.BlockSpec((256, 1), full),
          pl.BlockSpec(memory_space=pltpu.SMEM),
      ],
      out_specs=pl.BlockSpec((bl // 8, 128), lambda j: (j, 0)),
      out_shape=jax.ShapeDtypeStruct((e_pad // 8, 128), jnp.float32),
      compiler_params=pltpu.CompilerParams(
          dimension_semantics=("parallel",)
      ),
  )(ea_t, xs_pk, w1, b1, w2, b2, w3, b3, w4, b4, aa)


def _update_body(pa, pb, icnt, x, root, bias, aa, out):
  n = x.shape[0]
  s = pa[0, :n, :] + pa[1, :n, :] + pb[0, :n, :] + pb[1, :n, :]
  mean = s * icnt[...]
  v = mean + jnp.dot(x[...], root[...], preferred_element_type=jnp.float32)
  v = v + bias[...]
  out[...] = _prelu(v, aa[0])


def _update(pa, pb, icnt, x, root, bias, aa):
  n = x.shape[0]
  vm = pl.BlockSpec(memory_space=pltpu.VMEM)
  return pl.pallas_call(
      _update_body,
      in_specs=[vm, vm, vm, vm, vm, vm,
                pl.BlockSpec(memory_space=pltpu.SMEM)],
      out_specs=vm,
      out_shape=jax.ShapeDtypeStruct((n, 16), jnp.float32),
  )(pa, pb, icnt, x, root, bias, aa)


def _up_body(xin, w1, b1, w2, b2, w3, b3, aa, cparts, x0, icnt):
  h = jnp.dot(xin[...], w1[...], preferred_element_type=jnp.float32) + b1[...]
  h = _prelu(h, aa[0])
  h = jnp.dot(h, w2[...], preferred_element_type=jnp.float32) + b2[...]
  h = _prelu(h, aa[1])
  x0[...] = jnp.dot(h, w3[...], preferred_element_type=jnp.float32) + b3[...]
  n = xin.shape[0]
  cnt = jnp.maximum(
      cparts[0, 0, :n, :] + cparts[0, 1, :n, :]
      + cparts[1, 0, :n, :] + cparts[1, 1, :n, :], 1.0)
  icnt[...] = 1.0 / cnt


def _up(xin, w1, b1, w2, b2, w3, b3, aa, cparts):
  n = xin.shape[0]
  vm = pl.BlockSpec(memory_space=pltpu.VMEM)
  return pl.pallas_call(
      _up_body,
      in_specs=[vm, vm, vm, vm, vm, vm, vm,
                pl.BlockSpec(memory_space=pltpu.SMEM), vm],
      out_specs=(vm, vm),
      out_shape=(
          jax.ShapeDtypeStruct((n, 16), jnp.float32),
          jax.ShapeDtypeStruct((n, 16), jnp.float32),
      ),
  )(xin, w1, b1, w2, b2, w3, b3, aa, cparts)


def _heads_body(x, w1, b1, a1, w2, b2, a2, w3, b3, out):
  h = jnp.dot(x[...], w1[...], preferred_element_type=jnp.float32) + b1[...]
  h = jnp.where(h >= 0, h, a1[...] * h)
  h = jnp.dot(h, w2[...], preferred_element_type=jnp.float32) + b2[...]
  h = jnp.where(h >= 0, h, a2[...] * h)
  out[...] = jnp.dot(h, w3[...], preferred_element_type=jnp.float32) + b3[...]


def _heads(x, w1, b1, a1, w2, b2, a2, w3, b3):
  n = x.shape[0]
  vm = pl.BlockSpec(memory_space=pltpu.VMEM)
  return pl.pallas_call(
      _heads_body,
      in_specs=[vm] * 9,
      out_specs=vm,
      out_shape=jax.ShapeDtypeStruct((n, 6), jnp.float32),
  )(x, w1, b1, a1, w2, b2, a2, w3, b3)


def _block_diag(mats):
  rows = sum(m.shape[0] for m in mats)
  cols = sum(m.shape[1] for m in mats)
  out = jnp.zeros((rows, cols), jnp.float32)
  r = c = 0
  for m in mats:
    out = lax.dynamic_update_slice(out, m, (r, c))
    r += m.shape[0]
    c += m.shape[1]
  return out


def kernel(xxinc, xxcord, edgeidx, edgeattr, params):
  n = xxinc.shape[0]
  e = edgeattr.shape[0]
  per_tile_quantum = 2 * NW * CHUNK
  e_pad = -(-e // per_tile_quantum) * per_tile_quantum
  half = e_pad // 2
  n_chunks = half // (NW * CHUNK)

  src = edgeidx[0].astype(jnp.int32)
  dst = edgeidx[1].astype(jnp.int32)
  pad = e_pad - e
  src_pad = jnp.concatenate([src, jnp.zeros((pad,), jnp.int32)])
  dst_pad = jnp.concatenate([dst, jnp.full((pad,), n, jnp.int32)])
  src3 = [src_pad[i * half:(i + 1) * half].reshape(NW, n_chunks, CHUNK)
          for i in range(2)]
  dst3 = [dst_pad[i * half:(i + 1) * half].reshape(NW, n_chunks, CHUNK)
          for i in range(2)]

  bl = EDGE_BLOCK
  rpb = bl // 8
  ea_pad = jnp.concatenate(
      [edgeattr, jnp.zeros((pad, edgeattr.shape[1]), jnp.float32)])
  ea_perm = ea_pad.reshape(e_pad // bl, rpb, 8, 16).transpose(
      0, 2, 1, 3).reshape(e_pad, 16)
  ea_t = [ea_perm[i * half:(i + 1) * half].T for i in range(2)]

  ones_msg = jnp.ones((half // 8, 128), jnp.float32)
  cparts = jnp.stack([_sc_scatter(ones_msg, dst3[0], n_chunks, n),
                      _sc_scatter(ones_msg, dst3[1], n_chunks, n)])

  up = params["up"]
  xin = jnp.concatenate([xxinc, xxcord], axis=1)
  up_aa = jnp.stack([up["a"][0], up["a"][1]])
  x, icnt = _up(
      xin,
      up["lin"][0]["W"], up["lin"][0]["b"][None, :],
      up["lin"][1]["W"], up["lin"][1]["b"][None, :],
      up["lin"][2]["W"], up["lin"][2]["b"][None, :],
      up_aa, cparts)

  for s in params["steps"]:
    aggr = s["aggr"]
    aa = jnp.stack([s["aggr_a"][0], s["aggr_a"][1], s["aggr_a"][2]])
    ws = (aggr[0]["W"].T, aggr[0]["b"][:, None],
          aggr[1]["W"].T, aggr[1]["b"][:, None],
          aggr[2]["W"].T, aggr[2]["b"][:, None],
          aggr[3]["W"].T, aggr[3]["b"][:, None], aa)
    xs_a = _sc_gather(x, src3[0], n_chunks)
    xs_b = _sc_gather(x, src3[1], n_chunks)
    msg_a = _edge_msg(ea_t[0], xs_a, *ws)
    parts_a = _sc_scatter(msg_a, dst3[0], n_chunks, n)
    msg_b = _edge_msg(ea_t[1], xs_b, *ws)
    parts_b = _sc_scatter(msg_b, dst3[1], n_chunks, n)
    x = _update(parts_a, parts_b, icnt, x, s["root"], s["bias"][None, :],
                jnp.stack([s["out_a"]]))

  heads = params["heads"]
  w1 = jnp.concatenate([h["lin"][0]["W"] for h in heads], axis=1)
  b1 = jnp.concatenate([h["lin"][0]["b"] for h in heads])[None, :]
  a1 = jnp.concatenate(
      [jnp.full((8,), 1.0) * h["a"][0] for h in heads])[None, :]
  w2 = _block_diag([h["lin"][1]["W"] for h in heads])
  b2 = jnp.concatenate([h["lin"][1]["b"] for h in heads])[None, :]
  a2 = jnp.concatenate(
      [jnp.full((4,), 1.0) * h["a"][1] for h in heads])[None, :]
  w3 = _block_diag([h["lin"][2]["W"] for h in heads])
  b3 = jnp.concatenate([h["lin"][2]["b"] for h in heads])[None, :]
  return _heads(x, w1, b1, a1, w2, b2, a2, w3, b3)


# speedup vs baseline: 7.1227x; 1.0383x over previous
import functools

import jax
import jax.numpy as jnp
from jax import lax
from jax.experimental import pallas as pl
from jax.experimental.pallas import tpu as pltpu
from jax.experimental.pallas import tpu_sc as plsc

NC = 2
NS = 16
NW = NC * NS
CHUNK = 128
RING = 8

EDGE_BLOCK = 4096


@functools.partial(jax.jit, static_argnames=("n_chunks",))
def _sc_gather(x, idx3, n_chunks):
  per_tile = n_chunks * CHUNK
  per_pk = per_tile // 8
  e_pad = NW * per_tile
  cpk = CHUNK // 8
  mesh = plsc.VectorSubcoreMesh(core_axis_name="c", subcore_axis_name="s")

  @functools.partial(
      pl.kernel,
      out_type=jax.ShapeDtypeStruct((e_pad // 8, 128), jnp.float32),
      mesh=mesh,
      scratch_types=[
          pltpu.VMEM((n_chunks, CHUNK), jnp.int32),
          pltpu.VMEM((RING * CHUNK, 16), jnp.float32),
          pltpu.VMEM((per_pk, 128), jnp.float32),
          pltpu.SemaphoreType.DMA,
      ],
      compiler_params=pltpu.CompilerParams(use_tc_tiling_on_sc=False),
  )
  def gather(x_hbm, idx_hbm, out_hbm, idx_v, ring_v, pk_v, sem):
    wid = lax.axis_index("s") * NC + lax.axis_index("c")
    pltpu.sync_copy(idx_hbm.at[wid], idx_v)

    for j in range(RING):
      pltpu.make_async_copy(
          x_hbm.at[idx_v.at[j]], ring_v.at[pl.ds(j * CHUNK, CHUNK)], sem
      ).start()

    def body(j, carry):
      slot = lax.rem(j, RING)
      pltpu.make_async_copy(
          x_hbm.at[idx_v.at[j]], ring_v.at[pl.ds(slot * CHUNK, CHUNK)], sem
      ).wait()

      def rp(rr, c2):
        rbase = slot * CHUNK + 8 * rr
        for q in range(8):
          pk_v[j * cpk + rr, 16 * q:16 * (q + 1)] = ring_v[rbase + q, :]
        return c2

      lax.fori_loop(0, cpk, rp, 0)

      @pl.when(j + RING < n_chunks)
      def _():
        pltpu.make_async_copy(
            x_hbm.at[idx_v.at[j + RING]],
            ring_v.at[pl.ds(slot * CHUNK, CHUNK)], sem).start()

      return carry

    lax.fori_loop(0, n_chunks, body, 0)
    pltpu.sync_copy(pk_v, out_hbm.at[pl.ds(wid * per_pk, per_pk)])

  return gather(x, idx3)


@functools.partial(jax.jit, static_argnames=("n_chunks", "n_nodes"))
def _sc_scatter(msg_pk, idx3, n_chunks, n_nodes):
  per_tile = n_chunks * CHUNK
  per_pk = per_tile // 8
  cpk = CHUNK // 8
  zrows = -(-(n_nodes + 1) // NS)
  zrows = -(-zrows // 16) * 16
  tbl_rows = zrows * NS
  mesh = plsc.VectorSubcoreMesh(core_axis_name="c", subcore_axis_name="s")

  @functools.partial(
      pl.kernel,
      out_type=jax.ShapeDtypeStruct((NC, tbl_rows, 16), jnp.float32),
      mesh=mesh,
      scratch_types=[
          pltpu.VMEM((n_chunks, CHUNK), jnp.int32),
          pltpu.VMEM((per_pk, 128), jnp.float32),
          pltpu.VMEM((CHUNK, 16), jnp.float32),
          pltpu.VMEM((zrows, 16), jnp.float32),
          pltpu.VMEM_SHARED((tbl_rows, 16), jnp.float32),
          pltpu.SemaphoreType.DMA,
      ],
      compiler_params=pltpu.CompilerParams(use_tc_tiling_on_sc=False),
  )
  def scatter(msg_hbm, idx_hbm, out_hbm, idx_v, pk_v, grp_v, row_v, tbl,
              sem):
    cid = lax.axis_index("c")
    sid = lax.axis_index("s")
    wid = sid * NC + cid

    pltpu.make_async_copy(
        msg_hbm.at[pl.ds(wid * per_pk, per_pk)], pk_v, sem).start()

    def zbody(i, carry):
      row_v[i, :] = jnp.zeros((16,), jnp.float32)
      return carry

    lax.fori_loop(0, zrows, zbody, 0)
    pltpu.sync_copy(row_v, tbl.at[pl.ds(sid * zrows, zrows)])
    pltpu.sync_copy(idx_hbm.at[wid], idx_v)
    plsc.subcore_barrier()
    pltpu.make_async_copy(
        msg_hbm.at[pl.ds(wid * per_pk, per_pk)], pk_v, sem).wait()

    def sbody(g, carry):
      def unpack(rr, c2):
        for q in range(8):
          grp_v[8 * rr + q, :] = pk_v[g * cpk + rr, 16 * q:16 * (q + 1)]
        return c2

      lax.fori_loop(0, cpk, unpack, 0)
      pltpu.sync_copy(grp_v, tbl.at[idx_v.at[g]], add=True)
      return carry

    lax.fori_loop(0, n_chunks, sbody, 0)
    plsc.subcore_barrier()
    pltpu.sync_copy(
        tbl.at[pl.ds(sid * zrows, zrows)],
        out_hbm.at[cid, pl.ds(sid * zrows, zrows)],
    )

  return scatter(msg_pk, idx3)


def _prelu(x, a):
  return jnp.where(x >= 0, x, a * x)


def _edge_body(ea_t, xs_pk, w1, b1, w2, b2, w3, b3, w4, b4, aa, msg_pk):
  b = xs_pk.shape[0] * 8
  r = b // 8
  h = jnp.dot(w1[...], ea_t[...], preferred_element_type=jnp.float32) + b1[...]
  h = _prelu(h, aa[0])
  h = jnp.dot(w2[...], h, preferred_element_type=jnp.float32) + b2[...]
  h = _prelu(h, aa[1])
  h = jnp.dot(w3[...], h, preferred_element_type=jnp.float32) + b3[...]
  h = _prelu(h, aa[2])
  z = jnp.dot(w4[...], h, preferred_element_type=jnp.float32) + b4[...]
  xt = xs_pk[...].T
  xs_t = jnp.concatenate(
      [xt[16 * q:16 * (q + 1), :] for q in range(8)], axis=1)
  zz = z.reshape(16, 16, b)
  msg_t = jnp.sum(zz * xs_t[:, None, :], axis=0)
  m128 = jnp.concatenate(
      [msg_t[:, r * q:r * (q + 1)] for q in range(8)], axis=0)
  msg_pk[...] = m128.T


def _edge_msg(ea_t, xs_pk, w1, b1, w2, b2, w3, b3, w4, b4, aa):
  e_pad = xs_pk.shape[0] * 8
  grid = e_pad // EDGE_BLOCK
  bl = EDGE_BLOCK
  full = lambda j: (0, 0)
  return pl.pallas_call(
      _edge_body,
      grid=(grid,),
      in_specs=[
          pl.BlockSpec((16, bl), lambda j: (0, j)),
          pl.BlockSpec((bl // 8, 128), lambda j: (j, 0)),
          pl.BlockSpec((64, 16), full),
          pl.BlockSpec((64, 1), full),
          pl.BlockSpec((64, 64), full),
          pl.BlockSpec((64, 1), full),
          pl.BlockSpec((64, 64), full),
          pl.BlockSpec((64, 1), full),
          pl.BlockSpec((256, 64), full),
          pl.BlockSpec((256, 1), full),
          pl.BlockSpec(memory_space=pltpu.SMEM),
      ],
      out_specs=pl.BlockSpec((bl // 8, 128), lambda j: (j, 0)),
      out_shape=jax.ShapeDtypeStruct((e_pad // 8, 128), jnp.float32),
      compiler_params=pltpu.CompilerParams(
          dimension_semantics=("parallel",)
      ),
  )(ea_t, xs_pk, w1, b1, w2, b2, w3, b3, w4, b4, aa)


@functools.partial(jax.jit, static_argnames=())
def _sc_update(pa, pb, ca, cb, x_old, aux):
  tbl_rows = x_old.shape[0]
  rows = tbl_rows // NW
  mesh = plsc.VectorSubcoreMesh(core_axis_name="c", subcore_axis_name="s")

  @functools.partial(
      pl.kernel,
      out_type=jax.ShapeDtypeStruct((tbl_rows, 16), jnp.float32),
      mesh=mesh,
      scratch_types=[
          pltpu.VMEM((rows, 16), jnp.float32),
          pltpu.VMEM((4, rows, 16), jnp.float32),
          pltpu.VMEM((4, rows, 16), jnp.float32),
          pltpu.VMEM((18, 16), jnp.float32),
          pltpu.VMEM((rows, 16), jnp.float32),
      ],
      compiler_params=pltpu.CompilerParams(use_tc_tiling_on_sc=False),
  )
  def upd(pa_h, pb_h, ca_h, cb_h, x_h, aux_h, out_h, xo_v, p_v, c_v, aux_v,
          xn_v):
    wid = lax.axis_index("s") * NC + lax.axis_index("c")
    base = wid * rows
    pltpu.sync_copy(x_h.at[pl.ds(base, rows)], xo_v)
    pltpu.sync_copy(pa_h.at[0, pl.ds(base, rows)], p_v.at[0])
    pltpu.sync_copy(pa_h.at[1, pl.ds(base, rows)], p_v.at[1])
    pltpu.sync_copy(pb_h.at[0, pl.ds(base, rows)], p_v.at[2])
    pltpu.sync_copy(pb_h.at[1, pl.ds(base, rows)], p_v.at[3])
    pltpu.sync_copy(ca_h.at[0, pl.ds(base, rows)], c_v.at[0])
    pltpu.sync_copy(ca_h.at[1, pl.ds(base, rows)], c_v.at[1])
    pltpu.sync_copy(cb_h.at[0, pl.ds(base, rows)], c_v.at[2])
    pltpu.sync_copy(cb_h.at[1, pl.ds(base, rows)], c_v.at[3])
    pltpu.sync_copy(aux_h, aux_v)

    def body(i, carry):
      s = p_v[0, i, :] + p_v[1, i, :] + p_v[2, i, :] + p_v[3, i, :]
      cnt = c_v[0, i, :] + c_v[1, i, :] + c_v[2, i, :] + c_v[3, i, :]
      cnt = jnp.maximum(cnt, 1.0)
      v = s / cnt + aux_v[16, :]
      xo = xo_v[i, :]
      for c in range(16):
        v = v + xo[c] * aux_v[c, :]
      av = aux_v[17, :]
      xn_v[i, :] = jnp.where(v >= 0, v, av * v)
      return carry

    lax.fori_loop(0, rows, body, 0)
    pltpu.sync_copy(xn_v, out_h.at[pl.ds(base, rows)])

  return upd(pa, pb, ca, cb, x_old, aux)


def _up_body(xin, w1, b1, w2, b2, w3, b3, aa, x0):
  h = jnp.dot(xin[...], w1[...], preferred_element_type=jnp.float32) + b1[...]
  h = _prelu(h, aa[0])
  h = jnp.dot(h, w2[...], preferred_element_type=jnp.float32) + b2[...]
  h = _prelu(h, aa[1])
  x0[...] = jnp.dot(h, w3[...], preferred_element_type=jnp.float32) + b3[...]


def _up(xin, w1, b1, w2, b2, w3, b3, aa):
  n = xin.shape[0]
  vm = pl.BlockSpec(memory_space=pltpu.VMEM)
  return pl.pallas_call(
      _up_body,
      in_specs=[vm, vm, vm, vm, vm, vm, vm,
                pl.BlockSpec(memory_space=pltpu.SMEM)],
      out_specs=vm,
      out_shape=jax.ShapeDtypeStruct((n, 16), jnp.float32),
  )(xin, w1, b1, w2, b2, w3, b3, aa)


def _heads_body(x, w1, b1, a1, w2, b2, a2, w3, b3, out):
  h = jnp.dot(x[...], w1[...], preferred_element_type=jnp.float32) + b1[...]
  h = jnp.where(h >= 0, h, a1[...] * h)
  h = jnp.dot(h, w2[...], preferred_element_type=jnp.float32) + b2[...]
  h = jnp.where(h >= 0, h, a2[...] * h)
  out[...] = jnp.dot(h, w3[...], preferred_element_type=jnp.float32) + b3[...]


def _heads(x, w1, b1, a1, w2, b2, a2, w3, b3):
  n = x.shape[0]
  vm = pl.BlockSpec(memory_space=pltpu.VMEM)
  return pl.pallas_call(
      _heads_body,
      in_specs=[vm] * 9,
      out_specs=vm,
      out_shape=jax.ShapeDtypeStruct((n, 6), jnp.float32),
  )(x, w1, b1, a1, w2, b2, a2, w3, b3)


def _block_diag(mats):
  rows = sum(m.shape[0] for m in mats)
  cols = sum(m.shape[1] for m in mats)
  out = jnp.zeros((rows, cols), jnp.float32)
  r = c = 0
  for m in mats:
    out = lax.dynamic_update_slice(out, m, (r, c))
    r += m.shape[0]
    c += m.shape[1]
  return out


def kernel(xxinc, xxcord, edgeidx, edgeattr, params):
  n = xxinc.shape[0]
  e = edgeattr.shape[0]
  per_tile_quantum = 2 * NW * CHUNK
  e_pad = -(-e // per_tile_quantum) * per_tile_quantum
  half = e_pad // 2
  n_chunks = half // (NW * CHUNK)

  src = edgeidx[0].astype(jnp.int32)
  dst = edgeidx[1].astype(jnp.int32)
  pad = e_pad - e
  src_pad = jnp.concatenate([src, jnp.zeros((pad,), jnp.int32)])
  dst_pad = jnp.concatenate([dst, jnp.full((pad,), n, jnp.int32)])
  src3 = [src_pad[i * half:(i + 1) * half].reshape(NW, n_chunks, CHUNK)
          for i in range(2)]
  dst3 = [dst_pad[i * half:(i + 1) * half].reshape(NW, n_chunks, CHUNK)
          for i in range(2)]

  bl = EDGE_BLOCK
  rpb = bl // 8
  ea_pad = jnp.concatenate(
      [edgeattr, jnp.zeros((pad, edgeattr.shape[1]), jnp.float32)])
  ea_perm = ea_pad.reshape(e_pad // bl, rpb, 8, 16).transpose(
      0, 2, 1, 3).reshape(e_pad, 16)
  ea_t = [ea_perm[i * half:(i + 1) * half].T for i in range(2)]

  ones_msg = jnp.ones((half // 8, 128), jnp.float32)
  ca = _sc_scatter(ones_msg, dst3[0], n_chunks, n)
  cb = _sc_scatter(ones_msg, dst3[1], n_chunks, n)
  tbl_rows = ca.shape[1]

  up = params["up"]
  xin = jnp.concatenate([xxinc, xxcord], axis=1)
  up_aa = jnp.stack([up["a"][0], up["a"][1]])
  x0 = _up(
      xin,
      up["lin"][0]["W"], up["lin"][0]["b"][None, :],
      up["lin"][1]["W"], up["lin"][1]["b"][None, :],
      up["lin"][2]["W"], up["lin"][2]["b"][None, :],
      up_aa)
  x = jnp.concatenate(
      [x0, jnp.zeros((tbl_rows - n, 16), jnp.float32)])

  for s in params["steps"]:
    aggr = s["aggr"]
    aa = jnp.stack([s["aggr_a"][0], s["aggr_a"][1], s["aggr_a"][2]])
    ws = (aggr[0]["W"].T, aggr[0]["b"][:, None],
          aggr[1]["W"].T, aggr[1]["b"][:, None],
          aggr[2]["W"].T, aggr[2]["b"][:, None],
          aggr[3]["W"].T, aggr[3]["b"][:, None], aa)
    aux = jnp.concatenate(
        [s["root"], s["bias"][None, :],
         jnp.full((1, 16), 1.0, jnp.float32) * s["out_a"]])
    xs_a = _sc_gather(x, src3[0], n_chunks)
    xs_b = _sc_gather(x, src3[1], n_chunks)
    msg_a = _edge_msg(ea_t[0], xs_a, *ws)
    parts_a = _sc_scatter(msg_a, dst3[0], n_chunks, n)
    msg_b = _edge_msg(ea_t[1], xs_b, *ws)
    parts_b = _sc_scatter(msg_b, dst3[1], n_chunks, n)
    x = _sc_update(parts_a, parts_b, ca, cb, x, aux)

  x = x[:n, :]
  heads = params["heads"]
  w1 = jnp.concatenate([h["lin"][0]["W"] for h in heads], axis=1)
  b1 = jnp.concatenate([h["lin"][0]["b"] for h in heads])[None, :]
  a1 = jnp.concatenate(
      [jnp.full((8,), 1.0) * h["a"][0] for h in heads])[None, :]
  w2 = _block_diag([h["lin"][1]["W"] for h in heads])
  b2 = jnp.concatenate([h["lin"][1]["b"] for h in heads])[None, :]
  a2 = jnp.concatenate(
      [jnp.full((4,), 1.0) * h["a"][1] for h in heads])[None, :]
  w3 = _block_diag([h["lin"][2]["W"] for h in heads])
  b3 = jnp.concatenate([h["lin"][2]["b"] for h in heads])[None, :]
  return _heads(x, w1, b1, a1, w2, b2, a2, w3, b3)
